# Initial kernel scaffold; baseline (speedup 1.0000x reference)
import numpy as np
import jax
import jax.numpy as jnp
from jax import lax
from jax.experimental import pallas as pl
from jax.experimental.pallas import tpu as pltpu
from jax.experimental.pallas import tpu_sc as plsc

N = 10000
E = 320000
D = 128
ED = 16
H = 8
H1 = 16
H2 = 25
THRESH = 0.03
SCALE = float(1.0 / np.log(30.0))

NC = 2
NS = 16
L = 16

CE = 400
EA = E // NS
NR = N // NS

_f32 = jnp.float32


def _dg(vec, idx):
    dnums = lax.GatherDimensionNumbers(
        offset_dims=(), collapsed_slice_dims=(0,), start_index_map=(0,))
    return lax.gather(vec, idx[:, None], dnums, (1,),
                      mode=lax.GatherScatterMode.PROMISE_IN_BOUNDS)


def _leaky(v, slope):
    return jnp.where(v >= 0, v, slope * v)


def _make_conv_sc(CH, head_split):
    NV = CH // L
    mesh = plsc.VectorSubcoreMesh(core_axis_name="c", subcore_axis_name="s",
                                  num_cores=NC, num_subcores=NS)

    def body(src_hbm, dst_hbm, alt_hbm, ale_hbm, h_hbm, outp_hbm,
             src_v, dst_v, ale_v, g1_v, g2_v, dn_v, ex_v, h_v, zd_v,
             dn_sh, out_sh, sem):
        c = lax.axis_index("c")
        s = lax.axis_index("s")
        iota = lax.iota(jnp.int32, L)
        rot = (iota % 8) + 8
        lane_lo = iota < 8
        htbl = h_hbm.at[c] if head_split else h_hbm
        outp = outp_hbm.at[c]

        def _z16(i, _):
            ex_v[i, :] = jnp.zeros((L,), _f32)
            zd_v[i, :] = jnp.zeros((L,), _f32)
            return 0
        lax.fori_loop(0, NR, _z16, 0)

        def _zh(i, _):
            for v in range(NV):
                h_v[i, pl.ds(L * v, L)] = jnp.zeros((L,), _f32)
            return 0
        lax.fori_loop(0, CE, _zh, 0)

        pltpu.sync_copy(zd_v, dn_sh.at[pl.ds(s * NR, NR)])
        pltpu.sync_copy(h_v, out_sh.at[pl.ds(s * NR, CE)])
        pltpu.sync_copy(h_v.at[pl.ds(0, NR - CE)],
                        out_sh.at[pl.ds(s * NR + CE, NR - CE)])
        plsc.subcore_barrier()

        def chunkA(k, _):
            base = s * EA + k * CE
            cp1 = pltpu.async_copy(src_hbm.at[pl.ds(base, CE)], src_v, sem)
            cp2 = pltpu.async_copy(dst_hbm.at[pl.ds(base, CE)], dst_v, sem)
            cp3 = pltpu.async_copy(ale_hbm.at[pl.ds(base, CE)], ale_v, sem)
            cp1.wait()
            cp2.wait()
            cp3.wait()
            g1 = pltpu.async_copy(alt_hbm.at[src_v], g1_v, sem)
            g2 = pltpu.async_copy(alt_hbm.at[dst_v], g2_v, sem)
            g1.wait()
            g2.wait()

            def ebody(i, _):
                v = g1_v[i, :] + _dg(g2_v[i, :], rot) + ale_v[i, :]
                v = _leaky(v, 0.2) * SCALE
                ex = jnp.exp(v)
                ex_v[i, :] = jnp.where(lane_lo, ex, 0.0)
                return 0
            lax.fori_loop(0, CE, ebody, 0)
            pltpu.sync_copy(ex_v, dn_sh.at[dst_v], add=True)
            return 0
        lax.fori_loop(0, EA // CE, chunkA, 0)
        plsc.subcore_barrier()

        if head_split:
            nchB = EA // CE
        else:
            nchB = (E // (NC * NS)) // CE

        def chunkB(k, _):
            if head_split:
                base = s * EA + k * CE
            else:
                base = c * (E // NC) + s * (E // (NC * NS)) + k * CE
            cp1 = pltpu.async_copy(src_hbm.at[pl.ds(base, CE)], src_v, sem)
            cp2 = pltpu.async_copy(dst_hbm.at[pl.ds(base, CE)], dst_v, sem)
            cp3 = pltpu.async_copy(ale_hbm.at[pl.ds(base, CE)], ale_v, sem)
            cp1.wait()
            cp2.wait()
            cp3.wait()
            g1 = pltpu.async_copy(alt_hbm.at[src_v], g1_v, sem)
            g2 = pltpu.async_copy(alt_hbm.at[dst_v], g2_v, sem)
            g3 = pltpu.async_copy(dn_sh.at[dst_v], dn_v, sem)
            g4 = pltpu.async_copy(htbl.at[src_v], h_v, sem)
            g1.wait()
            g2.wait()
            g3.wait()
            g4.wait()

            def ebody(i, _):
                v = g1_v[i, :] + _dg(g2_v[i, :], rot) + ale_v[i, :]
                v = _leaky(v, 0.2) * SCALE
                ex = jnp.where(lane_lo, jnp.exp(v), 0.0)
                al = ex / (dn_v[i, :] + 1e-16)
                al = jnp.where(al < THRESH, 0.0, al)
                for v_i in range(NV):
                    if head_split:
                        col = (iota + L * v_i) // 28 + 4 * c
                    else:
                        col = jnp.full((L,), v_i, jnp.int32)
                    asel = _dg(al, col)
                    h_v[i, pl.ds(L * v_i, L)] = h_v[i, pl.ds(L * v_i, L)] * asel
                return 0
            lax.fori_loop(0, CE, ebody, 0)
            pltpu.sync_copy(h_v, out_sh.at[dst_v], add=True)
            return 0
        lax.fori_loop(0, nchB, chunkB, 0)
        plsc.subcore_barrier()

        pltpu.sync_copy(out_sh.at[pl.ds(s * NR, NR)], outp.at[pl.ds(s * NR, NR)])

    return pl.kernel(
        body,
        out_type=jax.ShapeDtypeStruct((NC, N, CH), _f32),
        mesh=mesh,
        scratch_types=[
            pltpu.VMEM((CE,), jnp.int32),
            pltpu.VMEM((CE,), jnp.int32),
            pltpu.VMEM((CE, L), _f32),
            pltpu.VMEM((CE, L), _f32),
            pltpu.VMEM((CE, L), _f32),
            pltpu.VMEM((CE, L), _f32),
            pltpu.VMEM((CE, L), _f32),
            pltpu.VMEM((CE, CH), _f32),
            pltpu.VMEM((NR, L), _f32),
            pltpu.VMEM_SHARED((N, L), _f32),
            pltpu.VMEM_SHARED((N, CH), _f32),
            pltpu.SemaphoreType.DMA,
        ],
    )


_conv1_sc = _make_conv_sc(D, head_split=False)
_conv2_sc = _make_conv_sc(112, head_split=True)


def _tk1_body(x_ref, w_ref, bsd_ref, h_ref, alt_ref):
    x = x_ref[...]
    h_ref[...] = jnp.dot(x, w_ref[...], preferred_element_type=_f32)
    alt_ref[...] = jnp.dot(x, bsd_ref[...], preferred_element_type=_f32)


def _tk2_body(ea_ref, b1_ref, b2_ref, a1_ref, a2_ref):
    ea = ea_ref[...]
    a1_ref[...] = jnp.dot(ea, b1_ref[...], preferred_element_type=_f32)
    a2_ref[...] = jnp.dot(ea, b2_ref[...], preferred_element_type=_f32)


def _bn(t, g, b):
    mu = jnp.mean(t, axis=0)
    var = jnp.mean((t - mu) ** 2, axis=0)
    return (t - mu) / jnp.sqrt(var + 1e-5) * g + b


def _tk3_body(p_ref, x_ref, f1w, f1b, g1, b1, f2w, f2b, g2, b2,
              w2a, w2b, bsd2, h2_ref, alt2_ref):
    x = x_ref[...]
    t = p_ref[0] + p_ref[1]
    t = _leaky(jnp.dot(t, f1w[...], preferred_element_type=_f32) + f1b[...],
               0.01) + x
    t = _bn(t, g1[...], b1[...])
    t = _leaky(jnp.dot(t, f2w[...], preferred_element_type=_f32) + f2b[...],
               0.01) + x
    t = _bn(t, g2[...], b2[...])
    h2_ref[0] = jnp.dot(t, w2a[...], preferred_element_type=_f32)
    h2_ref[1] = jnp.dot(t, w2b[...], preferred_element_type=_f32)
    alt2_ref[...] = jnp.dot(t, bsd2[...], preferred_element_type=_f32)


def _tk4_body(o2_ref, x_ref, f3a, f3b_, fc3b, f4w, f4b, f5w, f5b,
              g3, b3, g4, b4, y_ref):
    x = x_ref[...]
    t = (jnp.dot(o2_ref[0], f3a[...], preferred_element_type=_f32)
         + jnp.dot(o2_ref[1], f3b_[...], preferred_element_type=_f32)
         + fc3b[...])
    t = _leaky(t, 0.01) + x
    t = _bn(t, g3[...], b3[...])
    t = _leaky(jnp.dot(t, f4w[...], preferred_element_type=_f32) + f4b[...],
               0.01) + x
    t = _bn(t, g4[...], b4[...])
    y_ref[...] = jnp.dot(t, f5w[...], preferred_element_type=_f32) + f5b[...]


def kernel(x, edge_index, edge_attr, W1, asrc1, adst1, We1, ae1, W2, asrc2,
           adst2, We2, ae2, fc1_w, fc1_b, fc2_w, fc2_b, fc3_w, fc3_b, fc4_w,
           fc4_b, fc5_w, fc5_b, bn1_g, bn1_b, bn2_g, bn2_b, bn3_g, bn3_b,
           bn4_g, bn4_b):
    src = edge_index[0]
    dst = edge_index[1]

    Bsd1 = jnp.concatenate(
        [(W1.reshape(D, H, H1) * asrc1[None]).sum(-1),
         (W1.reshape(D, H, H1) * adst1[None]).sum(-1)], axis=1)
    Bsd2 = jnp.concatenate(
        [(W2.reshape(D, H, H2) * asrc2[None]).sum(-1),
         (W2.reshape(D, H, H2) * adst2[None]).sum(-1)], axis=1)
    Be1 = jnp.pad((We1.reshape(ED, H, H1) * ae1[None]).sum(-1),
                  ((0, 0), (0, 8)))
    Be2 = jnp.pad((We2.reshape(ED, H, H2) * ae2[None]).sum(-1),
                  ((0, 0), (0, 8)))
    W2p = jnp.pad(W2.reshape(D, H, H2), ((0, 0), (0, 0), (0, 28 - H2)))
    W2a = W2p[:, :4].reshape(D, 112)
    W2b = W2p[:, 4:].reshape(D, 112)
    f3p = jnp.pad(fc3_w.reshape(H, H2, D), ((0, 0), (0, 28 - H2), (0, 0)))
    f3a = f3p[:4].reshape(112, D)
    f3b = f3p[4:].reshape(112, D)

    h1, alt1 = pl.pallas_call(
        _tk1_body,
        out_shape=(jax.ShapeDtypeStruct((N, D), _f32),
                   jax.ShapeDtypeStruct((N, 16), _f32)),
    )(x, W1, Bsd1)

    BE = 3200
    ale1, ale2 = pl.pallas_call(
        _tk2_body,
        grid=(E // BE,),
        in_specs=[pl.BlockSpec((BE, ED), lambda i: (i, 0)),
                  pl.BlockSpec((ED, 16), lambda i: (0, 0)),
                  pl.BlockSpec((ED, 16), lambda i: (0, 0))],
        out_specs=[pl.BlockSpec((BE, 16), lambda i: (i, 0)),
                   pl.BlockSpec((BE, 16), lambda i: (i, 0))],
        out_shape=(jax.ShapeDtypeStruct((E, 16), _f32),
                   jax.ShapeDtypeStruct((E, 16), _f32)),
    )(edge_attr, Be1, Be2)

    outp1 = _conv1_sc(src, dst, alt1, ale1, h1)

    h2s, alt2 = pl.pallas_call(
        _tk3_body,
        out_shape=(jax.ShapeDtypeStruct((NC, N, 112), _f32),
                   jax.ShapeDtypeStruct((N, 16), _f32)),
    )(outp1, x, fc1_w, fc1_b, bn1_g, bn1_b, fc2_w, fc2_b, bn2_g, bn2_b,
      W2a, W2b, Bsd2)

    outp2 = _conv2_sc(src, dst, alt2, ale2, h2s)

    y = pl.pallas_call(
        _tk4_body,
        out_shape=jax.ShapeDtypeStruct((N, 1), _f32),
    )(outp2, x, f3a, f3b, fc3_b, fc4_w, fc4_b, fc5_w, fc5_b,
      bn3_g, bn3_b, bn4_g, bn4_b)

    return y.reshape(-1)


# speedup vs baseline: 19.0920x; 19.0920x over previous
import numpy as np
import jax
import jax.numpy as jnp
from jax import lax
from jax.experimental import pallas as pl
from jax.experimental.pallas import tpu as pltpu
from jax.experimental.pallas import tpu_sc as plsc

N = 10000
E = 320000
D = 128
ED = 16
H = 8
H1 = 16
H2 = 25
THRESH = 0.03
SCALE = float(1.0 / np.log(30.0))

NC = 2
NS = 16
L = 16

CE = 80
EA = E // NS
NP = 10240
NR = NP // NS

_f32 = jnp.float32


def _dg(vec, idx):
    dnums = lax.GatherDimensionNumbers(
        offset_dims=(), collapsed_slice_dims=(0,), start_index_map=(0,))
    return lax.gather(vec, idx[:, None], dnums, (1,),
                      mode=lax.GatherScatterMode.PROMISE_IN_BOUNDS)


def _leaky(v, slope):
    return jnp.where(v >= 0, v, slope * v)


def _make_conv_sc(CH, n_pass, chan_per_head):
    NV = CH // L
    HG = H // (NC * n_pass)
    mesh = plsc.VectorSubcoreMesh(core_axis_name="c", subcore_axis_name="s",
                                  num_cores=NC, num_subcores=NS)

    def body(src_hbm, dst_hbm, alt_hbm, ale_hbm, h_hbm, outp_hbm, dnh_hbm,
             src_v, dst_v, ale_v, g1_v, g2_v, dn_v, ex_v, h_v, zd_v,
             dn_sh, out_sh, sem):
        c = lax.axis_index("c")
        s = lax.axis_index("s")
        iota = lax.iota(jnp.int32, L)
        rot = (iota % 8) + 8
        lane_lo = iota < 8

        def _z16(i, _):
            ex_v[i, :] = jnp.zeros((L,), _f32)
            zd_v[i, :] = jnp.zeros((L,), _f32)
            return 0
        lax.fori_loop(0, NR, _z16, 0)

        def _zh(i, _):
            for v in range(NV):
                h_v[i, pl.ds(L * v, L)] = jnp.zeros((L,), _f32)
            return 0
        lax.fori_loop(0, CE, _zh, 0)

        pltpu.sync_copy(zd_v, dn_sh.at[pl.ds(s * NR, NR)])
        plsc.subcore_barrier()

        def chunkA(k, _):
            base = s * EA + k * CE
            cp1 = pltpu.async_copy(src_hbm.at[pl.ds(base, CE)], src_v, sem)
            cp2 = pltpu.async_copy(dst_hbm.at[pl.ds(base, CE)], dst_v, sem)
            cp3 = pltpu.async_copy(ale_hbm.at[pl.ds(base, CE)], ale_v, sem)
            cp1.wait()
            cp2.wait()
            cp3.wait()
            g1 = pltpu.async_copy(alt_hbm.at[src_v], g1_v, sem)
            g2 = pltpu.async_copy(alt_hbm.at[dst_v], g2_v, sem)
            g1.wait()
            g2.wait()

            def ebody(i, _):
                v = g1_v[i, :] + _dg(g2_v[i, :], rot) + ale_v[i, :]
                v = _leaky(v, 0.2) * SCALE
                ex = jnp.exp(v)
                ex_v[i, :] = jnp.where(lane_lo, ex, 0.0)
                return 0
            lax.fori_loop(0, CE, ebody, 0)
            pltpu.sync_copy(ex_v, dn_sh.at[dst_v], add=True)
            return 0
        lax.fori_loop(0, EA // CE, chunkA, 0)
        plsc.subcore_barrier()
        pltpu.sync_copy(dn_sh.at[pl.ds(s * NR, NR)],
                        dnh_hbm.at[c].at[pl.ds(s * NR, NR)])
        plsc.subcore_barrier()

        for p in range(n_pass):
            q = n_pass * c + p
            htbl = h_hbm.at[q]
            outp = outp_hbm.at[q]

            for r in range(0, NR, CE):
                pltpu.sync_copy(h_v, out_sh.at[pl.ds(s * NR + r, CE)])
            plsc.subcore_barrier()

            def chunkB(k, _):
                base = s * EA + k * CE
                cp1 = pltpu.async_copy(src_hbm.at[pl.ds(base, CE)], src_v, sem)
                cp2 = pltpu.async_copy(dst_hbm.at[pl.ds(base, CE)], dst_v, sem)
                cp3 = pltpu.async_copy(ale_hbm.at[pl.ds(base, CE)], ale_v, sem)
                cp1.wait()
                cp2.wait()
                cp3.wait()
                g1 = pltpu.async_copy(alt_hbm.at[src_v], g1_v, sem)
                g2 = pltpu.async_copy(alt_hbm.at[dst_v], g2_v, sem)
                g3 = pltpu.async_copy(dnh_hbm.at[c].at[dst_v], dn_v, sem)
                g4 = pltpu.async_copy(htbl.at[src_v], h_v, sem)
                g1.wait()
                g2.wait()
                g3.wait()
                g4.wait()

                def ebody(i, _):
                    v = g1_v[i, :] + _dg(g2_v[i, :], rot) + ale_v[i, :]
                    v = _leaky(v, 0.2) * SCALE
                    ex = jnp.where(lane_lo, jnp.exp(v), 0.0)
                    al = ex / (dn_v[i, :] + 1e-16)
                    al = jnp.where(al < THRESH, 0.0, al)
                    for v_i in range(NV):
                        if chan_per_head == L:
                            lhead = jnp.full((L,), v_i, jnp.int32)
                        else:
                            lhead = lax.div(
                                iota + L * v_i,
                                jnp.full((L,), chan_per_head, jnp.int32))
                        col = lhead + HG * q
                        asel = _dg(al, col)
                        h_v[i, pl.ds(L * v_i, L)] = (
                            h_v[i, pl.ds(L * v_i, L)] * asel)
                    return 0
                lax.fori_loop(0, CE, ebody, 0)
                pltpu.sync_copy(h_v, out_sh.at[dst_v], add=True)
                return 0
            lax.fori_loop(0, EA // CE, chunkB, 0)
            plsc.subcore_barrier()

            pltpu.sync_copy(out_sh.at[pl.ds(s * NR, NR)],
                            outp.at[pl.ds(s * NR, NR)])

            if p + 1 < n_pass:
                lax.fori_loop(0, CE, _zh, 0)
                plsc.subcore_barrier()

    return pl.kernel(
        body,
        out_type=(jax.ShapeDtypeStruct((NC * n_pass, NP, CH), _f32),
                  jax.ShapeDtypeStruct((NC, NP, L), _f32)),
        mesh=mesh,
        compiler_params=pltpu.CompilerParams(use_tc_tiling_on_sc=False),
        scratch_types=[
            pltpu.VMEM((CE,), jnp.int32),
            pltpu.VMEM((CE,), jnp.int32),
            pltpu.VMEM((CE, L), _f32),
            pltpu.VMEM((CE, L), _f32),
            pltpu.VMEM((CE, L), _f32),
            pltpu.VMEM((CE, L), _f32),
            pltpu.VMEM((CE, L), _f32),
            pltpu.VMEM((CE, CH), _f32),
            pltpu.VMEM((NR, L), _f32),
            pltpu.VMEM_SHARED((NP, L), _f32),
            pltpu.VMEM_SHARED((NP, CH), _f32),
            pltpu.SemaphoreType.DMA,
        ],
    )


_conv1_sc = _make_conv_sc(CH=64, n_pass=1, chan_per_head=16)
_conv2_sc = _make_conv_sc(CH=64, n_pass=2, chan_per_head=32)


def _tk1_body(x_ref, w1a_ref, w1b_ref, bsd_ref, h_ref, alt_ref):
    x = x_ref[...]
    h_ref[0] = jnp.dot(x, w1a_ref[...], preferred_element_type=_f32)
    h_ref[1] = jnp.dot(x, w1b_ref[...], preferred_element_type=_f32)
    alt_ref[...] = jnp.dot(x, bsd_ref[...], preferred_element_type=_f32)


def _tk2_body(ea_ref, b1_ref, b2_ref, a1_ref, a2_ref):
    ea = ea_ref[...]
    a1_ref[...] = jnp.dot(ea, b1_ref[...], preferred_element_type=_f32)
    a2_ref[...] = jnp.dot(ea, b2_ref[...], preferred_element_type=_f32)


def _bn(t, g, b):
    mu = jnp.mean(t, axis=0)
    var = jnp.mean((t - mu) ** 2, axis=0)
    return (t - mu) / jnp.sqrt(var + 1e-5) * g + b


def _tk3_body(p_ref, x_ref, f1a, f1b, f1bias, g1, b1, f2w, f2b, g2, b2,
              w2q0, w2q1, w2q2, w2q3, bsd2, h2_ref, alt2_ref):
    x = x_ref[...]
    pp = p_ref[...]
    t = (jnp.dot(pp[0, :N], f1a[...], preferred_element_type=_f32)
         + jnp.dot(pp[1, :N], f1b[...], preferred_element_type=_f32)
         + f1bias[...])
    t = _leaky(t, 0.01) + x
    t = _bn(t, g1[...], b1[...])
    t = _leaky(jnp.dot(t, f2w[...], preferred_element_type=_f32) + f2b[...],
               0.01) + x
    t = _bn(t, g2[...], b2[...])
    h2_ref[0] = jnp.dot(t, w2q0[...], preferred_element_type=_f32)
    h2_ref[1] = jnp.dot(t, w2q1[...], preferred_element_type=_f32)
    h2_ref[2] = jnp.dot(t, w2q2[...], preferred_element_type=_f32)
    h2_ref[3] = jnp.dot(t, w2q3[...], preferred_element_type=_f32)
    alt2_ref[...] = jnp.dot(t, bsd2[...], preferred_element_type=_f32)


def _tk4_body(o2_ref, x_ref, f3q0, f3q1, f3q2, f3q3, fc3b, f4w, f4b,
              f5w, f5b, g3, b3, g4, b4, y_ref):
    x = x_ref[...]
    o2 = o2_ref[...]
    t = (jnp.dot(o2[0, :N], f3q0[...], preferred_element_type=_f32)
         + jnp.dot(o2[1, :N], f3q1[...], preferred_element_type=_f32)
         + jnp.dot(o2[2, :N], f3q2[...], preferred_element_type=_f32)
         + jnp.dot(o2[3, :N], f3q3[...], preferred_element_type=_f32)
         + fc3b[...])
    t = _leaky(t, 0.01) + x
    t = _bn(t, g3[...], b3[...])
    t = _leaky(jnp.dot(t, f4w[...], preferred_element_type=_f32) + f4b[...],
               0.01) + x
    t = _bn(t, g4[...], b4[...])
    y_ref[...] = jnp.dot(t, f5w[...], preferred_element_type=_f32) + f5b[...]


def kernel(x, edge_index, edge_attr, W1, asrc1, adst1, We1, ae1, W2, asrc2,
           adst2, We2, ae2, fc1_w, fc1_b, fc2_w, fc2_b, fc3_w, fc3_b, fc4_w,
           fc4_b, fc5_w, fc5_b, bn1_g, bn1_b, bn2_g, bn2_b, bn3_g, bn3_b,
           bn4_g, bn4_b):
    src = edge_index[0]
    dst = edge_index[1]

    Bsd1 = jnp.concatenate(
        [(W1.reshape(D, H, H1) * asrc1[None]).sum(-1),
         (W1.reshape(D, H, H1) * adst1[None]).sum(-1)], axis=1)
    Bsd2 = jnp.concatenate(
        [(W2.reshape(D, H, H2) * asrc2[None]).sum(-1),
         (W2.reshape(D, H, H2) * adst2[None]).sum(-1)], axis=1)
    Be1 = jnp.pad((We1.reshape(ED, H, H1) * ae1[None]).sum(-1),
                  ((0, 0), (0, 8)))
    Be2 = jnp.pad((We2.reshape(ED, H, H2) * ae2[None]).sum(-1),
                  ((0, 0), (0, 8)))
    W1r = W1.reshape(D, H, H1)
    W1a = W1r[:, :4].reshape(D, 64)
    W1b = W1r[:, 4:].reshape(D, 64)
    f1a = fc1_w[:64]
    f1b = fc1_w[64:]
    W2p = jnp.pad(W2.reshape(D, H, H2), ((0, 0), (0, 0), (0, 32 - H2)))
    w2q = [W2p[:, 2 * q:2 * q + 2].reshape(D, 64) for q in range(4)]
    f3p = jnp.pad(fc3_w.reshape(H, H2, D), ((0, 0), (0, 32 - H2), (0, 0)))
    f3q = [f3p[2 * q:2 * q + 2].reshape(64, D) for q in range(4)]

    h1s, alt1 = pl.pallas_call(
        _tk1_body,
        out_shape=(jax.ShapeDtypeStruct((2, N, 64), _f32),
                   jax.ShapeDtypeStruct((N, 16), _f32)),
    )(x, W1a, W1b, Bsd1)

    BE = 3200
    ale1, ale2 = pl.pallas_call(
        _tk2_body,
        grid=(E // BE,),
        in_specs=[pl.BlockSpec((BE, ED), lambda i: (i, 0)),
                  pl.BlockSpec((ED, 16), lambda i: (0, 0)),
                  pl.BlockSpec((ED, 16), lambda i: (0, 0))],
        out_specs=[pl.BlockSpec((BE, 16), lambda i: (i, 0)),
                   pl.BlockSpec((BE, 16), lambda i: (i, 0))],
        out_shape=(jax.ShapeDtypeStruct((E, 16), _f32),
                   jax.ShapeDtypeStruct((E, 16), _f32)),
    )(edge_attr, Be1, Be2)

    outp1, _ = _conv1_sc(src, dst, alt1, ale1, h1s)

    h2s, alt2 = pl.pallas_call(
        _tk3_body,
        out_shape=(jax.ShapeDtypeStruct((4, N, 64), _f32),
                   jax.ShapeDtypeStruct((N, 16), _f32)),
    )(outp1, x, f1a, f1b, fc1_b, bn1_g, bn1_b, fc2_w, fc2_b, bn2_g, bn2_b,
      w2q[0], w2q[1], w2q[2], w2q[3], Bsd2)

    outp2, _ = _conv2_sc(src, dst, alt2, ale2, h2s)

    y = pl.pallas_call(
        _tk4_body,
        out_shape=jax.ShapeDtypeStruct((N, 1), _f32),
    )(outp2, x, f3q[0], f3q[1], f3q[2], f3q[3], fc3_b, fc4_w, fc4_b,
      fc5_w, fc5_b, bn3_g, bn3_b, bn4_g, bn4_b)

    return y.reshape(-1)


# speedup vs baseline: 26.0320x; 1.3635x over previous
import numpy as np
import jax
import jax.numpy as jnp
from jax import lax
from jax.experimental import pallas as pl
from jax.experimental.pallas import tpu as pltpu
from jax.experimental.pallas import tpu_sc as plsc

N = 10000
E = 320000
D = 128
ED = 16
H = 8
H1 = 16
H2 = 25
THRESH = 0.03
SCALE = float(1.0 / np.log(30.0))

NC = 2
NS = 16
L = 16

CE = 80
EA = E // NS
NP = 10240
NR = NP // NS

_f32 = jnp.float32


def _dg(vec, idx):
    dnums = lax.GatherDimensionNumbers(
        offset_dims=(), collapsed_slice_dims=(0,), start_index_map=(0,))
    return lax.gather(vec, idx[:, None], dnums, (1,),
                      mode=lax.GatherScatterMode.PROMISE_IN_BOUNDS)


def _leaky(v, slope):
    return jnp.where(v >= 0, v, slope * v)


def _make_conv_sc(CH, n_pass, chan_per_head):
    NV = CH // L
    HG = H // (NC * n_pass)
    mesh = plsc.VectorSubcoreMesh(core_axis_name="c", subcore_axis_name="s",
                                  num_cores=NC, num_subcores=NS)

    def body(src_hbm, dst_hbm, alt_hbm, ale_hbm, h_hbm, outp_hbm, dnh_hbm,
             src_v, dst_v, ale_v, g1_v, g2_v, dn_v, ex_v, h_v, zd_v,
             dn_sh, out_sh, sem):
        c = lax.axis_index("c")
        s = lax.axis_index("s")
        iota = lax.iota(jnp.int32, L)
        rot = (iota % 8) + 8
        lane_lo = iota < 8

        def _z16(i, _):
            ex_v[i, :] = jnp.zeros((L,), _f32)
            zd_v[i, :] = jnp.zeros((L,), _f32)
            return 0
        lax.fori_loop(0, NR, _z16, 0)

        def _zh(i, _):
            for v in range(NV):
                h_v[i, pl.ds(L * v, L)] = jnp.zeros((L,), _f32)
            return 0
        lax.fori_loop(0, CE, _zh, 0)

        pltpu.sync_copy(zd_v, dn_sh.at[pl.ds(s * NR, NR)])
        plsc.subcore_barrier()

        def chunkA(k, _):
            base = s * EA + k * CE
            cp1 = pltpu.async_copy(src_hbm.at[pl.ds(base, CE)], src_v, sem)
            cp2 = pltpu.async_copy(dst_hbm.at[pl.ds(base, CE)], dst_v, sem)
            cp3 = pltpu.async_copy(ale_hbm.at[pl.ds(base, CE)], ale_v, sem)
            cp1.wait()
            cp2.wait()
            cp3.wait()
            g1 = pltpu.async_copy(alt_hbm.at[src_v], g1_v, sem)
            g2 = pltpu.async_copy(alt_hbm.at[dst_v], g2_v, sem)
            g1.wait()
            g2.wait()

            @plsc.parallel_loop(0, CE, unroll=4)
            def ebody(i):
                v = g1_v[i, :] + _dg(g2_v[i, :], rot) + ale_v[i, :]
                v = _leaky(v, 0.2) * SCALE
                ex = jnp.exp(v)
                ex_v[i, :] = jnp.where(lane_lo, ex, 0.0)
            pltpu.sync_copy(ex_v, dn_sh.at[dst_v], add=True)
            return 0
        lax.fori_loop(0, EA // CE, chunkA, 0)
        plsc.subcore_barrier()
        pltpu.sync_copy(dn_sh.at[pl.ds(s * NR, NR)],
                        dnh_hbm.at[c].at[pl.ds(s * NR, NR)])
        plsc.subcore_barrier()

        for p in range(n_pass):
            q = n_pass * c + p
            htbl = h_hbm.at[q]
            outp = outp_hbm.at[q]

            for r in range(0, NR, CE):
                pltpu.sync_copy(h_v, out_sh.at[pl.ds(s * NR + r, CE)])
            plsc.subcore_barrier()

            def chunkB(k, _):
                base = s * EA + k * CE
                cp1 = pltpu.async_copy(src_hbm.at[pl.ds(base, CE)], src_v, sem)
                cp2 = pltpu.async_copy(dst_hbm.at[pl.ds(base, CE)], dst_v, sem)
                cp3 = pltpu.async_copy(ale_hbm.at[pl.ds(base, CE)], ale_v, sem)
                cp1.wait()
                cp2.wait()
                cp3.wait()
                g1 = pltpu.async_copy(alt_hbm.at[src_v], g1_v, sem)
                g2 = pltpu.async_copy(alt_hbm.at[dst_v], g2_v, sem)
                g3 = pltpu.async_copy(dnh_hbm.at[c].at[dst_v], dn_v, sem)
                g4 = pltpu.async_copy(htbl.at[src_v], h_v, sem)
                g1.wait()
                g2.wait()
                g3.wait()
                g4.wait()

                @plsc.parallel_loop(0, CE, unroll=2)
                def ebody(i):
                    v = g1_v[i, :] + _dg(g2_v[i, :], rot) + ale_v[i, :]
                    v = _leaky(v, 0.2) * SCALE
                    ex = jnp.where(lane_lo, jnp.exp(v), 0.0)
                    al = ex / (dn_v[i, :] + 1e-16)
                    al = jnp.where(al < THRESH, 0.0, al)
                    for v_i in range(NV):
                        if chan_per_head == L:
                            lhead = jnp.full((L,), v_i, jnp.int32)
                        else:
                            lhead = lax.div(
                                iota + L * v_i,
                                jnp.full((L,), chan_per_head, jnp.int32))
                        col = lhead + HG * q
                        asel = _dg(al, col)
                        h_v[i, pl.ds(L * v_i, L)] = (
                            h_v[i, pl.ds(L * v_i, L)] * asel)
                pltpu.sync_copy(h_v, out_sh.at[dst_v], add=True)
                return 0
            lax.fori_loop(0, EA // CE, chunkB, 0)
            plsc.subcore_barrier()

            pltpu.sync_copy(out_sh.at[pl.ds(s * NR, NR)],
                            outp.at[pl.ds(s * NR, NR)])

            if p + 1 < n_pass:
                lax.fori_loop(0, CE, _zh, 0)
                plsc.subcore_barrier()

    return pl.kernel(
        body,
        out_type=(jax.ShapeDtypeStruct((NC * n_pass, NP, CH), _f32),
                  jax.ShapeDtypeStruct((NC, NP, L), _f32)),
        mesh=mesh,
        compiler_params=pltpu.CompilerParams(use_tc_tiling_on_sc=False),
        scratch_types=[
            pltpu.VMEM((CE,), jnp.int32),
            pltpu.VMEM((CE,), jnp.int32),
            pltpu.VMEM((CE, L), _f32),
            pltpu.VMEM((CE, L), _f32),
            pltpu.VMEM((CE, L), _f32),
            pltpu.VMEM((CE, L), _f32),
            pltpu.VMEM((CE, L), _f32),
            pltpu.VMEM((CE, CH), _f32),
            pltpu.VMEM((NR, L), _f32),
            pltpu.VMEM_SHARED((NP, L), _f32),
            pltpu.VMEM_SHARED((NP, CH), _f32),
            pltpu.SemaphoreType.DMA,
        ],
    )


_conv1_sc = _make_conv_sc(CH=64, n_pass=1, chan_per_head=16)
_conv2_sc = _make_conv_sc(CH=64, n_pass=2, chan_per_head=32)


def _tk1_body(x_ref, w1a_ref, w1b_ref, bsd_ref, h_ref, alt_ref):
    x = x_ref[...]
    h_ref[0] = jnp.dot(x, w1a_ref[...], preferred_element_type=_f32)
    h_ref[1] = jnp.dot(x, w1b_ref[...], preferred_element_type=_f32)
    alt_ref[...] = jnp.dot(x, bsd_ref[...], preferred_element_type=_f32)


def _tk2_body(ea_ref, b1_ref, b2_ref, a1_ref, a2_ref):
    ea = ea_ref[...]
    a1_ref[...] = jnp.dot(ea, b1_ref[...], preferred_element_type=_f32)
    a2_ref[...] = jnp.dot(ea, b2_ref[...], preferred_element_type=_f32)


def _bn(t, g, b):
    mu = jnp.mean(t, axis=0)
    var = jnp.mean((t - mu) ** 2, axis=0)
    return (t - mu) / jnp.sqrt(var + 1e-5) * g + b


def _tk3_body(p_ref, x_ref, f1a, f1b, f1bias, g1, b1, f2w, f2b, g2, b2,
              w2q0, w2q1, w2q2, w2q3, bsd2, h2_ref, alt2_ref):
    x = x_ref[...]
    pp = p_ref[...]
    t = (jnp.dot(pp[0, :N], f1a[...], preferred_element_type=_f32)
         + jnp.dot(pp[1, :N], f1b[...], preferred_element_type=_f32)
         + f1bias[...])
    t = _leaky(t, 0.01) + x
    t = _bn(t, g1[...], b1[...])
    t = _leaky(jnp.dot(t, f2w[...], preferred_element_type=_f32) + f2b[...],
               0.01) + x
    t = _bn(t, g2[...], b2[...])
    h2_ref[0] = jnp.dot(t, w2q0[...], preferred_element_type=_f32)
    h2_ref[1] = jnp.dot(t, w2q1[...], preferred_element_type=_f32)
    h2_ref[2] = jnp.dot(t, w2q2[...], preferred_element_type=_f32)
    h2_ref[3] = jnp.dot(t, w2q3[...], preferred_element_type=_f32)
    alt2_ref[...] = jnp.dot(t, bsd2[...], preferred_element_type=_f32)


def _tk4_body(o2_ref, x_ref, f3q0, f3q1, f3q2, f3q3, fc3b, f4w, f4b,
              f5w, f5b, g3, b3, g4, b4, y_ref):
    x = x_ref[...]
    o2 = o2_ref[...]
    t = (jnp.dot(o2[0, :N], f3q0[...], preferred_element_type=_f32)
         + jnp.dot(o2[1, :N], f3q1[...], preferred_element_type=_f32)
         + jnp.dot(o2[2, :N], f3q2[...], preferred_element_type=_f32)
         + jnp.dot(o2[3, :N], f3q3[...], preferred_element_type=_f32)
         + fc3b[...])
    t = _leaky(t, 0.01) + x
    t = _bn(t, g3[...], b3[...])
    t = _leaky(jnp.dot(t, f4w[...], preferred_element_type=_f32) + f4b[...],
               0.01) + x
    t = _bn(t, g4[...], b4[...])
    y_ref[...] = jnp.dot(t, f5w[...], preferred_element_type=_f32) + f5b[...]


def kernel(x, edge_index, edge_attr, W1, asrc1, adst1, We1, ae1, W2, asrc2,
           adst2, We2, ae2, fc1_w, fc1_b, fc2_w, fc2_b, fc3_w, fc3_b, fc4_w,
           fc4_b, fc5_w, fc5_b, bn1_g, bn1_b, bn2_g, bn2_b, bn3_g, bn3_b,
           bn4_g, bn4_b):
    src = edge_index[0]
    dst = edge_index[1]

    Bsd1 = jnp.concatenate(
        [(W1.reshape(D, H, H1) * asrc1[None]).sum(-1),
         (W1.reshape(D, H, H1) * adst1[None]).sum(-1)], axis=1)
    Bsd2 = jnp.concatenate(
        [(W2.reshape(D, H, H2) * asrc2[None]).sum(-1),
         (W2.reshape(D, H, H2) * adst2[None]).sum(-1)], axis=1)
    Be1 = jnp.pad((We1.reshape(ED, H, H1) * ae1[None]).sum(-1),
                  ((0, 0), (0, 8)))
    Be2 = jnp.pad((We2.reshape(ED, H, H2) * ae2[None]).sum(-1),
                  ((0, 0), (0, 8)))
    W1r = W1.reshape(D, H, H1)
    W1a = W1r[:, :4].reshape(D, 64)
    W1b = W1r[:, 4:].reshape(D, 64)
    f1a = fc1_w[:64]
    f1b = fc1_w[64:]
    W2p = jnp.pad(W2.reshape(D, H, H2), ((0, 0), (0, 0), (0, 32 - H2)))
    w2q = [W2p[:, 2 * q:2 * q + 2].reshape(D, 64) for q in range(4)]
    f3p = jnp.pad(fc3_w.reshape(H, H2, D), ((0, 0), (0, 32 - H2), (0, 0)))
    f3q = [f3p[2 * q:2 * q + 2].reshape(64, D) for q in range(4)]

    h1s, alt1 = pl.pallas_call(
        _tk1_body,
        out_shape=(jax.ShapeDtypeStruct((2, N, 64), _f32),
                   jax.ShapeDtypeStruct((N, 16), _f32)),
    )(x, W1a, W1b, Bsd1)

    BE = 3200
    ale1, ale2 = pl.pallas_call(
        _tk2_body,
        grid=(E // BE,),
        in_specs=[pl.BlockSpec((BE, ED), lambda i: (i, 0)),
                  pl.BlockSpec((ED, 16), lambda i: (0, 0)),
                  pl.BlockSpec((ED, 16), lambda i: (0, 0))],
        out_specs=[pl.BlockSpec((BE, 16), lambda i: (i, 0)),
                   pl.BlockSpec((BE, 16), lambda i: (i, 0))],
        out_shape=(jax.ShapeDtypeStruct((E, 16), _f32),
                   jax.ShapeDtypeStruct((E, 16), _f32)),
    )(edge_attr, Be1, Be2)

    outp1, _ = _conv1_sc(src, dst, alt1, ale1, h1s)

    h2s, alt2 = pl.pallas_call(
        _tk3_body,
        out_shape=(jax.ShapeDtypeStruct((4, N, 64), _f32),
                   jax.ShapeDtypeStruct((N, 16), _f32)),
    )(outp1, x, f1a, f1b, fc1_b, bn1_g, bn1_b, fc2_w, fc2_b, bn2_g, bn2_b,
      w2q[0], w2q[1], w2q[2], w2q[3], Bsd2)

    outp2, _ = _conv2_sc(src, dst, alt2, ale2, h2s)

    y = pl.pallas_call(
        _tk4_body,
        out_shape=jax.ShapeDtypeStruct((N, 1), _f32),
    )(outp2, x, f3q[0], f3q[1], f3q[2], f3q[3], fc3_b, fc4_w, fc4_b,
      fc5_w, fc5_b, bn3_g, bn3_b, bn4_g, bn4_b)

    return y.reshape(-1)


# speedup vs baseline: 41.6958x; 1.6017x over previous
import numpy as np
import jax
import jax.numpy as jnp
from jax import lax
from jax.experimental import pallas as pl
from jax.experimental.pallas import tpu as pltpu
from jax.experimental.pallas import tpu_sc as plsc

N = 10000
E = 320000
D = 128
ED = 16
H = 8
H1 = 16
H2 = 25
THRESH = 0.03
SCALE = float(1.0 / np.log(30.0))

NC = 2
NS = 16
L = 16

CE = 80
EA = E // NS
NP = 10240
NR = NP // NS

_f32 = jnp.float32


def _dg(vec, idx):
    dnums = lax.GatherDimensionNumbers(
        offset_dims=(), collapsed_slice_dims=(0,), start_index_map=(0,))
    return lax.gather(vec, idx[:, None], dnums, (1,),
                      mode=lax.GatherScatterMode.PROMISE_IN_BOUNDS)


def _leaky(v, slope):
    return jnp.where(v >= 0, v, slope * v)


def _make_conv_sc(CH, n_pass, chan_per_head):
    NV = CH // L
    HG = H // (NC * n_pass)
    NCH = EA // CE
    assert NCH % 2 == 0
    mesh = plsc.VectorSubcoreMesh(core_axis_name="c", subcore_axis_name="s",
                                  num_cores=NC, num_subcores=NS)

    def body(src_hbm, dst_hbm, alt_hbm, ale_hbm, h_hbm, outp_hbm, dnh_hbm,
             src_a, dst_a, ale_a, g1_a, g2_a, dn_a, h_a,
             src_b, dst_b, ale_b, g1_b, g2_b, dn_b, h_b,
             ex_v, zd_v, dn_sh, out_sh, ls_a, ls_b, gs_a, gs_b):
        c = lax.axis_index("c")
        s = lax.axis_index("s")
        iota = lax.iota(jnp.int32, L)
        rot = (iota % 8) + 8
        lane_lo = iota < 8
        ebase = s * EA

        bufs = ((src_a, dst_a, ale_a, g1_a, g2_a, dn_a, h_a, ls_a, gs_a),
                (src_b, dst_b, ale_b, g1_b, g2_b, dn_b, h_b, ls_b, gs_b))

        def lin_issue(base, pb):
            sv, dv, av, _, _, _, _, ls, _ = pb
            pltpu.async_copy(src_hbm.at[pl.ds(base, CE)], sv, ls)
            pltpu.async_copy(dst_hbm.at[pl.ds(base, CE)], dv, ls)
            pltpu.async_copy(ale_hbm.at[pl.ds(base, CE)], av, ls)

        def lin_wait(pb):
            sv, dv, av, _, _, _, _, ls, _ = pb
            pltpu.make_async_copy(src_hbm.at[pl.ds(0, CE)], sv, ls).wait()
            pltpu.make_async_copy(dst_hbm.at[pl.ds(0, CE)], dv, ls).wait()
            pltpu.make_async_copy(ale_hbm.at[pl.ds(0, CE)], av, ls).wait()

        def _z16(i, _):
            ex_v[i, :] = jnp.zeros((L,), _f32)
            zd_v[i, :] = jnp.zeros((L,), _f32)
            return 0
        lax.fori_loop(0, NR, _z16, 0)

        def _zh(i, _):
            for v in range(NV):
                h_a[i, pl.ds(L * v, L)] = jnp.zeros((L,), _f32)
            return 0
        lax.fori_loop(0, CE, _zh, 0)

        pltpu.sync_copy(zd_v, dn_sh.at[pl.ds(s * NR, NR)])
        plsc.subcore_barrier()

        def gathA_issue(pb):
            sv, dv, _, g1, g2, _, _, _, gs = pb
            pltpu.async_copy(alt_hbm.at[sv], g1, gs)
            pltpu.async_copy(alt_hbm.at[dv], g2, gs)

        def gathA_wait(pb):
            _, _, _, g1, g2, _, _, _, gs = pb
            pltpu.make_async_copy(alt_hbm.at[pl.ds(0, CE)], g1, gs).wait()
            pltpu.make_async_copy(alt_hbm.at[pl.ds(0, CE)], g2, gs).wait()

        def computeA(pb):
            _, dv, av, g1, g2, _, _, _, _ = pb

            @plsc.parallel_loop(0, CE, unroll=4)
            def ebody(i):
                v = g1[i, :] + _dg(g2[i, :], rot) + av[i, :]
                v = _leaky(v, 0.2) * SCALE
                ex = jnp.exp(v)
                ex_v[i, :] = jnp.where(lane_lo, ex, 0.0)
            pltpu.sync_copy(ex_v, dn_sh.at[dv], add=True)

        lin_issue(ebase, bufs[0])
        lin_wait(bufs[0])
        gathA_issue(bufs[0])
        lin_issue(ebase + CE, bufs[1])

        def pairA(k2, _):
            for j in range(2):
                pb, nb = bufs[j], bufs[1 - j]
                gathA_wait(pb)
                lin_wait(nb)
                gathA_issue(nb)
                computeA(pb)
                lin_issue(ebase + (2 * k2 + j + 2) * CE, pb)
            return 0
        lax.fori_loop(0, NCH // 2 - 1, pairA, 0)
        gathA_wait(bufs[0])
        lin_wait(bufs[1])
        gathA_issue(bufs[1])
        computeA(bufs[0])
        gathA_wait(bufs[1])
        computeA(bufs[1])
        plsc.subcore_barrier()
        pltpu.sync_copy(dn_sh.at[pl.ds(s * NR, NR)],
                        dnh_hbm.at[c].at[pl.ds(s * NR, NR)])
        plsc.subcore_barrier()

        for p in range(n_pass):
            q = n_pass * c + p
            htbl = h_hbm.at[q]
            outp = outp_hbm.at[q]

            for r in range(0, NR, CE):
                pltpu.sync_copy(h_a, out_sh.at[pl.ds(s * NR + r, CE)])
            plsc.subcore_barrier()

            def gathB_issue(pb):
                sv, dv, _, g1, g2, dn, hv, _, gs = pb
                pltpu.async_copy(alt_hbm.at[sv], g1, gs)
                pltpu.async_copy(alt_hbm.at[dv], g2, gs)
                pltpu.async_copy(dnh_hbm.at[c].at[dv], dn, gs)
                pltpu.async_copy(htbl.at[sv], hv, gs)

            def gathB_wait(pb):
                _, _, _, g1, g2, dn, hv, _, gs = pb
                pltpu.make_async_copy(alt_hbm.at[pl.ds(0, CE)], g1, gs).wait()
                pltpu.make_async_copy(alt_hbm.at[pl.ds(0, CE)], g2, gs).wait()
                pltpu.make_async_copy(dnh_hbm.at[c].at[pl.ds(0, CE)], dn,
                                      gs).wait()
                pltpu.make_async_copy(htbl.at[pl.ds(0, CE)], hv, gs).wait()

            def computeB(pb):
                _, dv, av, g1, g2, dn, hv, _, _ = pb

                @plsc.parallel_loop(0, CE, unroll=2)
                def ebody(i):
                    v = g1[i, :] + _dg(g2[i, :], rot) + av[i, :]
                    v = _leaky(v, 0.2) * SCALE
                    ex = jnp.where(lane_lo, jnp.exp(v), 0.0)
                    al = ex / (dn[i, :] + 1e-16)
                    al = jnp.where(al < THRESH, 0.0, al)
                    for v_i in range(NV):
                        if chan_per_head == L:
                            lhead = jnp.full((L,), v_i, jnp.int32)
                        else:
                            lhead = lax.div(
                                iota + L * v_i,
                                jnp.full((L,), chan_per_head, jnp.int32))
                        col = lhead + HG * q
                        asel = _dg(al, col)
                        hv[i, pl.ds(L * v_i, L)] = (
                            hv[i, pl.ds(L * v_i, L)] * asel)
                pltpu.sync_copy(hv, out_sh.at[dv], add=True)

            lin_issue(ebase, bufs[1])
            lin_wait(bufs[1])
            gathB_issue(bufs[1])
            lin_issue(ebase + CE, bufs[0])

            def pairB(k2, _):
                for j in range(2):
                    pb, nb = bufs[1 - j], bufs[j]
                    gathB_wait(pb)
                    lin_wait(nb)
                    gathB_issue(nb)
                    computeB(pb)
                    lin_issue(ebase + (2 * k2 + j + 2) * CE, pb)
                return 0
            lax.fori_loop(0, NCH // 2 - 1, pairB, 0)
            gathB_wait(bufs[1])
            lin_wait(bufs[0])
            gathB_issue(bufs[0])
            computeB(bufs[1])
            gathB_wait(bufs[0])
            computeB(bufs[0])
            plsc.subcore_barrier()

            pltpu.sync_copy(out_sh.at[pl.ds(s * NR, NR)],
                            outp.at[pl.ds(s * NR, NR)])

            if p + 1 < n_pass:
                lax.fori_loop(0, CE, _zh, 0)
                plsc.subcore_barrier()

    vb = [pltpu.VMEM((CE,), jnp.int32), pltpu.VMEM((CE,), jnp.int32),
          pltpu.VMEM((CE, L), _f32), pltpu.VMEM((CE, L), _f32),
          pltpu.VMEM((CE, L), _f32), pltpu.VMEM((CE, L), _f32),
          pltpu.VMEM((CE, CH), _f32)]
    return pl.kernel(
        body,
        out_type=(jax.ShapeDtypeStruct((NC * n_pass, NP, CH), _f32),
                  jax.ShapeDtypeStruct((NC, NP, L), _f32)),
        mesh=mesh,
        compiler_params=pltpu.CompilerParams(use_tc_tiling_on_sc=False),
        scratch_types=vb + vb + [
            pltpu.VMEM((CE, L), _f32),
            pltpu.VMEM((NR, L), _f32),
            pltpu.VMEM_SHARED((NP, L), _f32),
            pltpu.VMEM_SHARED((NP, CH), _f32),
            pltpu.SemaphoreType.DMA,
            pltpu.SemaphoreType.DMA,
            pltpu.SemaphoreType.DMA,
            pltpu.SemaphoreType.DMA,
        ],
    )


_conv1_sc = _make_conv_sc(CH=64, n_pass=1, chan_per_head=16)
_conv2_sc = _make_conv_sc(CH=64, n_pass=2, chan_per_head=32)


def _tk1_body(x_ref, w1a_ref, w1b_ref, bsd_ref, h_ref, alt_ref):
    x = x_ref[...]
    h_ref[0] = jnp.dot(x, w1a_ref[...], preferred_element_type=_f32)
    h_ref[1] = jnp.dot(x, w1b_ref[...], preferred_element_type=_f32)
    alt_ref[...] = jnp.dot(x, bsd_ref[...], preferred_element_type=_f32)


def _tk2_body(ea_ref, b1_ref, b2_ref, a1_ref, a2_ref):
    ea = ea_ref[...]
    a1_ref[...] = jnp.dot(ea, b1_ref[...], preferred_element_type=_f32)
    a2_ref[...] = jnp.dot(ea, b2_ref[...], preferred_element_type=_f32)


def _bn(t, g, b):
    mu = jnp.mean(t, axis=0)
    var = jnp.mean((t - mu) ** 2, axis=0)
    return (t - mu) / jnp.sqrt(var + 1e-5) * g + b


def _tk3_body(p_ref, x_ref, f1a, f1b, f1bias, g1, b1, f2w, f2b, g2, b2,
              w2q0, w2q1, w2q2, w2q3, bsd2, h2_ref, alt2_ref):
    x = x_ref[...]
    pp = p_ref[...]
    t = (jnp.dot(pp[0, :N], f1a[...], preferred_element_type=_f32)
         + jnp.dot(pp[1, :N], f1b[...], preferred_element_type=_f32)
         + f1bias[...])
    t = _leaky(t, 0.01) + x
    t = _bn(t, g1[...], b1[...])
    t = _leaky(jnp.dot(t, f2w[...], preferred_element_type=_f32) + f2b[...],
               0.01) + x
    t = _bn(t, g2[...], b2[...])
    h2_ref[0] = jnp.dot(t, w2q0[...], preferred_element_type=_f32)
    h2_ref[1] = jnp.dot(t, w2q1[...], preferred_element_type=_f32)
    h2_ref[2] = jnp.dot(t, w2q2[...], preferred_element_type=_f32)
    h2_ref[3] = jnp.dot(t, w2q3[...], preferred_element_type=_f32)
    alt2_ref[...] = jnp.dot(t, bsd2[...], preferred_element_type=_f32)


def _tk4_body(o2_ref, x_ref, f3q0, f3q1, f3q2, f3q3, fc3b, f4w, f4b,
              f5w, f5b, g3, b3, g4, b4, y_ref):
    x = x_ref[...]
    o2 = o2_ref[...]
    t = (jnp.dot(o2[0, :N], f3q0[...], preferred_element_type=_f32)
         + jnp.dot(o2[1, :N], f3q1[...], preferred_element_type=_f32)
         + jnp.dot(o2[2, :N], f3q2[...], preferred_element_type=_f32)
         + jnp.dot(o2[3, :N], f3q3[...], preferred_element_type=_f32)
         + fc3b[...])
    t = _leaky(t, 0.01) + x
    t = _bn(t, g3[...], b3[...])
    t = _leaky(jnp.dot(t, f4w[...], preferred_element_type=_f32) + f4b[...],
               0.01) + x
    t = _bn(t, g4[...], b4[...])
    y_ref[...] = jnp.dot(t, f5w[...], preferred_element_type=_f32) + f5b[...]


def kernel(x, edge_index, edge_attr, W1, asrc1, adst1, We1, ae1, W2, asrc2,
           adst2, We2, ae2, fc1_w, fc1_b, fc2_w, fc2_b, fc3_w, fc3_b, fc4_w,
           fc4_b, fc5_w, fc5_b, bn1_g, bn1_b, bn2_g, bn2_b, bn3_g, bn3_b,
           bn4_g, bn4_b):
    src = edge_index[0]
    dst = edge_index[1]

    Bsd1 = jnp.concatenate(
        [(W1.reshape(D, H, H1) * asrc1[None]).sum(-1),
         (W1.reshape(D, H, H1) * adst1[None]).sum(-1)], axis=1)
    Bsd2 = jnp.concatenate(
        [(W2.reshape(D, H, H2) * asrc2[None]).sum(-1),
         (W2.reshape(D, H, H2) * adst2[None]).sum(-1)], axis=1)
    Be1 = jnp.pad((We1.reshape(ED, H, H1) * ae1[None]).sum(-1),
                  ((0, 0), (0, 8)))
    Be2 = jnp.pad((We2.reshape(ED, H, H2) * ae2[None]).sum(-1),
                  ((0, 0), (0, 8)))
    W1r = W1.reshape(D, H, H1)
    W1a = W1r[:, :4].reshape(D, 64)
    W1b = W1r[:, 4:].reshape(D, 64)
    f1a = fc1_w[:64]
    f1b = fc1_w[64:]
    W2p = jnp.pad(W2.reshape(D, H, H2), ((0, 0), (0, 0), (0, 32 - H2)))
    w2q = [W2p[:, 2 * q:2 * q + 2].reshape(D, 64) for q in range(4)]
    f3p = jnp.pad(fc3_w.reshape(H, H2, D), ((0, 0), (0, 32 - H2), (0, 0)))
    f3q = [f3p[2 * q:2 * q + 2].reshape(64, D) for q in range(4)]

    h1s, alt1 = pl.pallas_call(
        _tk1_body,
        out_shape=(jax.ShapeDtypeStruct((2, N, 64), _f32),
                   jax.ShapeDtypeStruct((N, 16), _f32)),
    )(x, W1a, W1b, Bsd1)

    BE = 3200
    ale1, ale2 = pl.pallas_call(
        _tk2_body,
        grid=(E // BE,),
        in_specs=[pl.BlockSpec((BE, ED), lambda i: (i, 0)),
                  pl.BlockSpec((ED, 16), lambda i: (0, 0)),
                  pl.BlockSpec((ED, 16), lambda i: (0, 0))],
        out_specs=[pl.BlockSpec((BE, 16), lambda i: (i, 0)),
                   pl.BlockSpec((BE, 16), lambda i: (i, 0))],
        out_shape=(jax.ShapeDtypeStruct((E, 16), _f32),
                   jax.ShapeDtypeStruct((E, 16), _f32)),
    )(edge_attr, Be1, Be2)

    outp1, _ = _conv1_sc(src, dst, alt1, ale1, h1s)

    h2s, alt2 = pl.pallas_call(
        _tk3_body,
        out_shape=(jax.ShapeDtypeStruct((4, N, 64), _f32),
                   jax.ShapeDtypeStruct((N, 16), _f32)),
    )(outp1, x, f1a, f1b, fc1_b, bn1_g, bn1_b, fc2_w, fc2_b, bn2_g, bn2_b,
      w2q[0], w2q[1], w2q[2], w2q[3], Bsd2)

    outp2, _ = _conv2_sc(src, dst, alt2, ale2, h2s)

    y = pl.pallas_call(
        _tk4_body,
        out_shape=jax.ShapeDtypeStruct((N, 1), _f32),
    )(outp2, x, f3q[0], f3q[1], f3q[2], f3q[3], fc3_b, fc4_w, fc4_b,
      fc5_w, fc5_b, bn3_g, bn3_b, bn4_g, bn4_b)

    return y.reshape(-1)
